# Initial kernel scaffold; baseline (speedup 1.0000x reference)
import jax
import jax.numpy as jnp
from jax.experimental import pallas as pl


def kernel(x, preds, Wq1, gq1, bq1, Wq2, gq2, bq2, Wk1, gk1, bk1, Wk2, gk2, bk2, Wv, gv, bv, Wo, go, bo):
    raise NotImplementedError("write your pallas kernel here")


# speedup vs baseline: 2.8590x; 2.8590x over previous
import jax
import jax.numpy as jnp
from jax import lax
from jax.experimental import pallas as pl

_B, _C, _D, _H, _W = 2, 128, 8, 16, 32
_HW = _H * _W
_N = _D * _HW
_T = 64
_EPS = 1e-5
_NEG = -1e30
_BLK = 512


def _mm(w, ys):
    return [
        lax.dot_general(w, y, (((1,), (0,)), ((), ())),
                        preferred_element_type=jnp.float32)
        for y in ys
    ]


def _bn_relu(ys, g, b):
    n = float(len(ys) * ys[0].shape[1])
    mean = sum(jnp.sum(y, axis=1, keepdims=True) for y in ys) / n
    var = sum(jnp.sum((y - mean) ** 2, axis=1, keepdims=True) for y in ys) / n
    inv = g / jnp.sqrt(var + _EPS)
    return [jnp.maximum((y - mean) * inv + b, 0.0) for y in ys]


def _fwd_kernel(x_ref, p_ref,
                wq1, gq1, bq1, wq2, gq2, bq2,
                wk1, gk1, bk1, wk2, gk2, bk2,
                wv, gv, bv, wo, go, bo,
                out_ref):
    xs = [x_ref[0], x_ref[1]]

    kfs = []
    for b in range(_B):
        pb = p_ref[b]
        m = jnp.max(pb, axis=0, keepdims=True)
        e = jnp.exp(pb - m)
        psm = e / jnp.sum(e, axis=0, keepdims=True)
        selp = jnp.max(psm, axis=0, keepdims=True)
        dio = lax.broadcasted_iota(jnp.int32, (_D, _HW), 0)
        cand = jnp.where(psm >= selp, dio, _D)
        amax = jnp.min(cand, axis=0, keepdims=True)
        onehot = dio == amax
        segmax = jnp.max(jnp.where(onehot, selp, _NEG), axis=1, keepdims=True)
        selsm = jnp.max(jnp.where(onehot, segmax, _NEG), axis=0, keepdims=True)
        ex = jnp.exp(selp - selsm)
        segsum = jnp.sum(jnp.where(onehot, ex, 0.0), axis=1, keepdims=True)
        denom = jnp.max(jnp.where(onehot, segsum, _NEG), axis=0, keepdims=True)
        wgt = ex / denom
        ohf = jnp.where(onehot, 1.0, 0.0)
        xb = xs[b]
        self_f = None
        for d in range(_D):
            t = xb[:, d * _HW:(d + 1) * _HW] * ohf[d:d + 1, :]
            self_f = t if self_f is None else self_f + t
        wf = self_f * wgt
        kfs.append(jnp.concatenate(
            [xb[:, d * _HW:(d + 1) * _HW] + ohf[d:d + 1, :] * wf
             for d in range(_D)], axis=1))

    q = _bn_relu(_mm(wq2[...], _bn_relu(_mm(wq1[...], xs), gq1[...], bq1[...])),
                 gq2[...], bq2[...])
    k = _bn_relu(_mm(wk2[...], _bn_relu(_mm(wk1[...], kfs), gk1[...], bk1[...])),
                 gk2[...], bk2[...])
    v = _bn_relu(_mm(wv[...], kfs), gv[...], bv[...])

    scale = _T ** -0.5
    ctxs = []
    for b in range(_B):
        qb, kb, vb = q[b], k[b], v[b]
        cols = []
        for blk in range(_N // _BLK):
            qblk = qb[:, blk * _BLK:(blk + 1) * _BLK]
            s = lax.dot_general(qblk, kb, (((0,), (0,)), ((), ())),
                                preferred_element_type=jnp.float32) * scale
            sm = jnp.max(s, axis=1, keepdims=True)
            se = jnp.exp(s - sm)
            p = se / jnp.sum(se, axis=1, keepdims=True)
            cols.append(lax.dot_general(vb, p, (((1,), (1,)), ((), ())),
                                        preferred_element_type=jnp.float32))
        ctxs.append(jnp.concatenate(cols, axis=1))

    outs = _bn_relu(_mm(wo[...], ctxs), go[...], bo[...])
    out_ref[0, :, :] = outs[0]
    out_ref[1, :, :] = outs[1]


def kernel(x, preds, Wq1, gq1, bq1, Wq2, gq2, bq2, Wk1, gk1, bk1,
           Wk2, gk2, bk2, Wv, gv, bv, Wo, go, bo):
    xr = x.reshape(_B, _C, _N)
    pr = preds.reshape(_B, _D, _HW)
    v2 = lambda a: a.reshape(-1, 1)
    out = pl.pallas_call(
        _fwd_kernel,
        out_shape=jax.ShapeDtypeStruct((_B, _C, _N), x.dtype),
    )(xr, pr,
      Wq1, v2(gq1), v2(bq1), Wq2, v2(gq2), v2(bq2),
      Wk1, v2(gk1), v2(bk1), Wk2, v2(gk2), v2(bk2),
      Wv, v2(gv), v2(bv), Wo, v2(go), v2(bo))
    return out.reshape(_B, _C, _D, _H, _W)


# speedup vs baseline: 3.5997x; 1.2591x over previous
import jax
import jax.numpy as jnp
from jax import lax
from jax.experimental import pallas as pl

_B, _C, _D, _H, _W = 2, 128, 8, 16, 32
_HW = _H * _W
_N = _D * _HW
_T = 64
_EPS = 1e-5
_NEG = -1e30
_BLK = 512


def _mm(w, ys):
    return [
        lax.dot_general(w, y, (((1,), (0,)), ((), ())),
                        preferred_element_type=jnp.float32)
        for y in ys
    ]


def _bn_relu(ys, g, b):
    n = float(len(ys) * ys[0].shape[1])
    mean = sum(jnp.sum(y, axis=1, keepdims=True) for y in ys) / n
    var = sum(jnp.sum((y - mean) ** 2, axis=1, keepdims=True) for y in ys) / n
    inv = g / jnp.sqrt(var + _EPS)
    return [jnp.maximum((y - mean) * inv + b, 0.0) for y in ys]


def _fwd_kernel(x_ref, p_ref,
                wq1, gq1, bq1, wq2, gq2, bq2,
                wk1, gk1, bk1, wk2, gk2, bk2,
                wv, gv, bv, wo, go, bo,
                out_ref):
    xs = [x_ref[0], x_ref[1]]

    kfs = []
    for b in range(_B):
        pb = p_ref[b]
        m = jnp.max(pb, axis=0, keepdims=True)
        e = jnp.exp(pb - m)
        psm = e / jnp.sum(e, axis=0, keepdims=True)
        selp = jnp.max(psm, axis=0, keepdims=True)
        dio = lax.broadcasted_iota(jnp.int32, (_D, _HW), 0)
        cand = jnp.where(psm >= selp, dio, _D)
        amax = jnp.min(cand, axis=0, keepdims=True)
        onehot = dio == amax
        segmax = jnp.max(jnp.where(onehot, selp, _NEG), axis=1, keepdims=True)
        selsm = jnp.max(jnp.where(onehot, segmax, _NEG), axis=0, keepdims=True)
        ex = jnp.exp(selp - selsm)
        segsum = jnp.sum(jnp.where(onehot, ex, 0.0), axis=1, keepdims=True)
        denom = jnp.max(jnp.where(onehot, segsum, _NEG), axis=0, keepdims=True)
        wgt = ex / denom
        ohf = jnp.where(onehot, 1.0, 0.0)
        xb = xs[b]
        self_f = None
        for d in range(_D):
            t = xb[:, d * _HW:(d + 1) * _HW] * ohf[d:d + 1, :]
            self_f = t if self_f is None else self_f + t
        wf = self_f * wgt
        kfs.append(jnp.concatenate(
            [xb[:, d * _HW:(d + 1) * _HW] + ohf[d:d + 1, :] * wf
             for d in range(_D)], axis=1))

    q = _bn_relu(_mm(wq2[...], _bn_relu(_mm(wq1[...], xs), gq1[...], bq1[...])),
                 gq2[...], bq2[...])
    k = _bn_relu(_mm(wk2[...], _bn_relu(_mm(wk1[...], kfs), gk1[...], bk1[...])),
                 gk2[...], bk2[...])
    v = _bn_relu(_mm(wv[...], kfs), gv[...], bv[...])

    scale = _T ** -0.5
    ctxs = []
    for b in range(_B):
        qb, kb, vb = q[b] * scale, k[b], v[b]
        vb_aug = jnp.concatenate(
            [vb, jnp.ones((1, _N), jnp.float32)], axis=0)
        cols = []
        for blk in range(_N // _BLK):
            qblk = qb[:, blk * _BLK:(blk + 1) * _BLK]
            s = lax.dot_general(qblk, kb, (((0,), (0,)), ((), ())),
                                preferred_element_type=jnp.float32)
            sm = jnp.max(s, axis=1, keepdims=True)
            se = jnp.exp(s - sm)
            ca = lax.dot_general(vb_aug, se, (((1,), (1,)), ((), ())),
                                 preferred_element_type=jnp.float32)
            cols.append(ca[:_T, :] / ca[_T:_T + 1, :])
        ctxs.append(jnp.concatenate(cols, axis=1))

    outs = _bn_relu(_mm(wo[...], ctxs), go[...], bo[...])
    out_ref[0, :, :] = outs[0]
    out_ref[1, :, :] = outs[1]


def kernel(x, preds, Wq1, gq1, bq1, Wq2, gq2, bq2, Wk1, gk1, bk1,
           Wk2, gk2, bk2, Wv, gv, bv, Wo, go, bo):
    xr = x.reshape(_B, _C, _N)
    pr = preds.reshape(_B, _D, _HW)
    v2 = lambda a: a.reshape(-1, 1)
    out = pl.pallas_call(
        _fwd_kernel,
        out_shape=jax.ShapeDtypeStruct((_B, _C, _N), x.dtype),
    )(xr, pr,
      Wq1, v2(gq1), v2(bq1), Wq2, v2(gq2), v2(bq2),
      Wk1, v2(gk1), v2(bk1), Wk2, v2(gk2), v2(bk2),
      Wv, v2(gv), v2(bv), Wo, v2(go), v2(bo))
    return out.reshape(_B, _C, _D, _H, _W)
